# baseline (speedup 1.0000x reference)
import functools

import jax
import jax.numpy as jnp
from jax import lax
from jax.experimental import pallas as pl
from jax.experimental.pallas import tpu as pltpu
from jax.experimental.pallas import tpu_sc as plsc

DIM = 128
LANES = 16
CHUNK = 64
RING = 4


def _rsqrt(x):
    bits = lax.bitcast_convert_type(x, jnp.int32)
    magic = jnp.full(x.shape, 0x5F3759DF, jnp.int32)
    y = lax.bitcast_convert_type(magic - (bits >> 1), jnp.float32)
    for _ in range(3):
        y = y * (1.5 - 0.5 * x * y * y)
    return y


def _make_sc_kernel(n_rows, n_workers, num_cores, n_ctx):
    rows_per_w = n_rows // n_workers
    n_chunks = rows_per_w // CHUNK
    assert n_chunks % RING == 0 and n_chunks >= 2 * RING
    n_blocks = n_chunks // RING
    mesh = plsc.VectorSubcoreMesh(core_axis_name="c", subcore_axis_name="s")

    @functools.partial(
        pl.kernel,
        out_type=jax.ShapeDtypeStruct((n_rows, DIM), jnp.float32),
        mesh=mesh,
        compiler_params=pltpu.CompilerParams(needs_layout_passes=False),
        scratch_types=(
            [pltpu.VMEM((4, CHUNK), jnp.int32) for _ in range(RING)]
            + [pltpu.VMEM((CHUNK, DIM), jnp.float32) for _ in range(RING)]
            + [pltpu.VMEM((CHUNK, DIM), jnp.float32) for _ in range(RING)]
            + [
                pltpu.VMEM((DIM,), jnp.float32),
                pltpu.VMEM((DIM,), jnp.float32),
                pltpu.VMEM((DIM, LANES), jnp.float32),
                pltpu.VMEM((DIM, LANES), jnp.float32),
                pltpu.VMEM((n_ctx, DIM), jnp.float32),
                pltpu.SemaphoreType.DMA((RING,)),
                pltpu.SemaphoreType.DMA((RING,)),
                pltpu.SemaphoreType.DMA((RING,)),
            ]
        ),
    )
    def run(idx_h, ttab, stab, dtab, ptab, gam_h, bet_h, out_h,
            ix0, ix1, ix2, ix3, rw0, rw1, rw2, rw3,
            ob0, ob1, ob2, ob3,
            gam, bet, gskew, bskew, pos_v, sem_g, sem_a, sem_s):
        idxs = (ix0, ix1, ix2, ix3)
        rows = (rw0, rw1, rw2, rw3)
        outb = (ob0, ob1, ob2, ob3)
        wid = lax.axis_index("s") * num_cores + lax.axis_index("c")
        cbase = wid * n_chunks
        rbase = wid * rows_per_w
        pltpu.sync_copy(gam_h, gam)
        pltpu.sync_copy(bet_h, bet)
        pltpu.sync_copy(ptab, pos_v)
        lane = lax.iota(jnp.int32, LANES)

        def skew_body(c, carry):
            cv = (lane + c) & (DIM - 1)
            gskew[c, :] = plsc.load_gather(gam, [cv])
            bskew[c, :] = plsc.load_gather(bet, [cv])
            return carry

        lax.fori_loop(0, DIM, skew_body, 0)

        def idx_load(ci, j):
            pltpu.sync_copy(idx_h.at[cbase + ci], idxs[j])

        def gather_plain(j):
            pltpu.async_copy(ttab.at[idxs[j].at[0]], rows[j], sem_g.at[j])

        def wait_plain(j):
            pltpu.make_async_copy(
                ttab.at[idxs[j].at[0]], rows[j], sem_g.at[j]).wait()

        def gather_adds(j):
            pltpu.async_copy(stab.at[idxs[j].at[1]], rows[j], sem_a.at[j],
                             add=True)
            pltpu.async_copy(dtab.at[idxs[j].at[2]], rows[j], sem_a.at[j],
                             add=True)

        def wait_adds(j):
            pltpu.make_async_copy(
                stab.at[idxs[j].at[1]], rows[j], sem_a.at[j]).wait()
            pltpu.make_async_copy(
                dtab.at[idxs[j].at[2]], rows[j], sem_a.at[j]).wait()

        def store(ci, j):
            pltpu.async_copy(
                outb[j], out_h.at[pl.ds(rbase + ci * CHUNK, CHUNK)],
                sem_s.at[j])

        def wait_store(ci, j):
            pltpu.make_async_copy(
                outb[j], out_h.at[pl.ds(rbase + ci * CHUNK, CHUNK)],
                sem_s.at[j]).wait()

        def compute(j):
            buf = rows[j]
            obuf = outb[j]

            def group_body(g, carry):
                row_idx = g * LANES + lane
                pidx = idxs[j][3, pl.ds(g * LANES, LANES)]

                def col_body(c, sc):
                    s, ss = sc
                    cv = (lane + c) & (DIM - 1)
                    x = (plsc.load_gather(buf, [row_idx, cv])
                         + plsc.load_gather(pos_v, [pidx, cv]))
                    return s + x, ss + x * x

                zeros = jnp.zeros((LANES,), jnp.float32)
                s, ss = lax.fori_loop(0, DIM, col_body, (zeros, zeros),
                                      unroll=8)
                mu = s * (1.0 / DIM)
                var = ss * (1.0 / DIM) - mu * mu
                rstd = _rsqrt(var + 1e-5)

                def col_body2(c, carry2):
                    cv = (lane + c) & (DIM - 1)
                    x = (plsc.load_gather(buf, [row_idx, cv])
                         + plsc.load_gather(pos_v, [pidx, cv]))
                    y = (x - mu) * rstd * gskew[c, :] + bskew[c, :]
                    plsc.store_scatter(obuf, [row_idx, cv], y)
                    return carry2

                lax.fori_loop(0, DIM, col_body2, 0, unroll=8)
                return carry

            lax.fori_loop(0, CHUNK // LANES, group_body, 0)

        idx_load(0, 0)
        idx_load(1, 1)
        gather_plain(0)
        wait_plain(0)
        gather_adds(0)
        gather_plain(1)

        def block(k, carry):
            for p in range(RING):
                i = k * RING + p
                j1 = (p + 1) % RING
                j2 = (p + 2) % RING

                @pl.when(i + 1 < n_chunks)
                def _():
                    wait_plain(j1)
                    gather_adds(j1)

                @pl.when(i >= RING - 2)
                def _():
                    wait_store(i - (RING - 2), j2)

                @pl.when(i + 2 < n_chunks)
                def _():
                    idx_load(i + 2, j2)
                    gather_plain(j2)

                wait_adds(p)
                compute(p)
                store(i, p)
            return carry

        lax.fori_loop(0, n_blocks, block, 0)
        for q in range(n_chunks - (RING - 2), n_chunks):
            wait_store(q, q % RING)

    return run


def kernel(token, segment, domain, position, token_table, segment_table,
           domain_table, pos_table, gamma, beta):
    b, l = token.shape
    n = b * l
    info = plsc.get_sparse_core_info()
    n_workers = info.num_cores * info.num_subcores
    run = _make_sc_kernel(n, n_workers, info.num_cores, pos_table.shape[0])
    ids = jnp.stack([
        token.reshape(n).astype(jnp.int32),
        segment.reshape(n).astype(jnp.int32),
        domain.reshape(n).astype(jnp.int32),
        position.reshape(n).astype(jnp.int32),
    ])
    idx_h = ids.reshape(4, n // CHUNK, CHUNK).transpose(1, 0, 2)
    out = run(idx_h, token_table, segment_table, domain_table, pos_table,
              gamma, beta)
    return out.reshape(b, l, DIM)


# speedup vs baseline: 2.3244x; 2.3244x over previous
import functools

import jax
import jax.numpy as jnp
from jax import lax
from jax.experimental import pallas as pl
from jax.experimental.pallas import tpu as pltpu
from jax.experimental.pallas import tpu_sc as plsc

DIM = 128
LANES = 16
NVREG = DIM // LANES
CHUNK = 128
RING = 5


def _rsqrt(x):
    bits = lax.bitcast_convert_type(x, jnp.int32)
    magic = jnp.full(x.shape, 0x5F3759DF, jnp.int32)
    y = lax.bitcast_convert_type(magic - (bits >> 1), jnp.float32)
    for _ in range(3):
        y = y * (1.5 - 0.5 * x * y * y)
    return y


def _make_sc_kernel(n_rows, n_workers, num_cores):
    rows_per_w = n_rows // n_workers
    n_chunks = rows_per_w // CHUNK
    assert n_chunks % RING == 0 and n_chunks >= 2 * RING
    n_blocks = n_chunks // RING
    mesh = plsc.VectorSubcoreMesh(core_axis_name="c", subcore_axis_name="s")

    @functools.partial(
        pl.kernel,
        out_type=jax.ShapeDtypeStruct((n_rows, DIM), jnp.float32),
        mesh=mesh,
        compiler_params=pltpu.CompilerParams(needs_layout_passes=False),
        scratch_types=(
            [pltpu.VMEM((4, CHUNK), jnp.int32) for _ in range(RING)]
            + [pltpu.VMEM((CHUNK, DIM), jnp.float32) for _ in range(RING)]
            + [
                pltpu.VMEM((DIM,), jnp.float32),
                pltpu.VMEM((DIM,), jnp.float32),
                pltpu.VMEM((CHUNK,), jnp.float32),
                pltpu.VMEM((CHUNK,), jnp.float32),
                pltpu.SemaphoreType.DMA((RING,)),
                pltpu.SemaphoreType.DMA((RING,)),
                pltpu.SemaphoreType.DMA((RING,)),
            ]
        ),
    )
    def run(idx_h, ttab, stab, dtab, ptab, gam_h, bet_h, out_h,
            ix0, ix1, ix2, ix3, ix4, rw0, rw1, rw2, rw3, rw4,
            gam, bet, mu_buf, rs_buf, sem_g, sem_a, sem_s):
        idxs = (ix0, ix1, ix2, ix3, ix4)
        rows = (rw0, rw1, rw2, rw3, rw4)
        wid = lax.axis_index("s") * num_cores + lax.axis_index("c")
        cbase = wid * n_chunks
        rbase = wid * rows_per_w
        pltpu.sync_copy(gam_h, gam)
        pltpu.sync_copy(bet_h, bet)
        lane = lax.iota(jnp.int32, LANES)

        def idx_load(ci, j):
            pltpu.sync_copy(idx_h.at[cbase + ci], idxs[j])

        def gather_plain(j):
            pltpu.async_copy(ttab.at[idxs[j].at[0]], rows[j], sem_g.at[j])

        def wait_plain(j):
            pltpu.make_async_copy(
                ttab.at[idxs[j].at[0]], rows[j], sem_g.at[j]).wait()

        def gather_adds(j):
            pltpu.async_copy(stab.at[idxs[j].at[1]], rows[j], sem_a.at[j],
                             add=True)
            pltpu.async_copy(dtab.at[idxs[j].at[2]], rows[j], sem_a.at[j],
                             add=True)
            pltpu.async_copy(ptab.at[idxs[j].at[3]], rows[j], sem_a.at[j],
                             add=True)

        def wait_adds(j):
            pltpu.make_async_copy(
                stab.at[idxs[j].at[1]], rows[j], sem_a.at[j]).wait()
            pltpu.make_async_copy(
                dtab.at[idxs[j].at[2]], rows[j], sem_a.at[j]).wait()
            pltpu.make_async_copy(
                ptab.at[idxs[j].at[3]], rows[j], sem_a.at[j]).wait()

        def store(ci, j):
            pltpu.async_copy(
                rows[j], out_h.at[pl.ds(rbase + ci * CHUNK, CHUNK)],
                sem_s.at[j])

        def wait_store(ci, j):
            pltpu.make_async_copy(
                rows[j], out_h.at[pl.ds(rbase + ci * CHUNK, CHUNK)],
                sem_s.at[j]).wait()

        def compute(j):
            buf = rows[j]

            def group_body(g, carry):
                row_idx = g * LANES + lane

                def col_body(c, sc):
                    s, ss = sc
                    cv = (lane + c) & (DIM - 1)
                    col = plsc.load_gather(buf, [row_idx, cv])
                    return s + col, ss + col * col

                zeros = jnp.zeros((LANES,), jnp.float32)
                s, ss = lax.fori_loop(0, DIM, col_body, (zeros, zeros),
                                      unroll=8)
                mu = s * (1.0 / DIM)
                var = ss * (1.0 / DIM) - mu * mu
                rstd = _rsqrt(var + 1e-5)
                mu_buf[pl.ds(g * LANES, LANES)] = mu
                rs_buf[pl.ds(g * LANES, LANES)] = rstd
                return carry

            lax.fori_loop(0, CHUNK // LANES, group_body, 0)

            gs = [gam[pl.ds(LANES * k, LANES)] for k in range(NVREG)]
            bs = [bet[pl.ds(LANES * k, LANES)] for k in range(NVREG)]

            def row_body(r, c):
                rv = jnp.full((LANES,), r, jnp.int32)
                mu = plsc.load_gather(mu_buf, [rv])
                rstd = plsc.load_gather(rs_buf, [rv])
                for k in range(NVREG):
                    x = buf[r, pl.ds(LANES * k, LANES)]
                    buf[r, pl.ds(LANES * k, LANES)] = (
                        (x - mu) * rstd * gs[k] + bs[k])
                return c

            lax.fori_loop(0, CHUNK, row_body, 0, unroll=2)

        idx_load(0, 0)
        idx_load(1, 1)
        gather_plain(0)
        wait_plain(0)
        gather_adds(0)
        gather_plain(1)

        def block(k, carry):
            for p in range(RING):
                i = k * RING + p
                j1 = (p + 1) % RING
                j2 = (p + 2) % RING

                @pl.when(i + 1 < n_chunks)
                def _():
                    wait_plain(j1)
                    gather_adds(j1)

                @pl.when(i >= 3)
                def _():
                    wait_store(i - 3, j2)

                @pl.when(i + 2 < n_chunks)
                def _():
                    idx_load(i + 2, j2)
                    gather_plain(j2)

                wait_adds(p)
                compute(p)
                store(i, p)
            return carry

        lax.fori_loop(0, n_blocks, block, 0)
        for p in range(RING - 3, RING):
            wait_store(n_chunks - RING + p, p)

    return run


def kernel(token, segment, domain, position, token_table, segment_table,
           domain_table, pos_table, gamma, beta):
    b, l = token.shape
    n = b * l
    info = plsc.get_sparse_core_info()
    n_workers = info.num_cores * info.num_subcores
    run = _make_sc_kernel(n, n_workers, info.num_cores)
    ids = jnp.stack([
        token.reshape(n).astype(jnp.int32),
        segment.reshape(n).astype(jnp.int32),
        domain.reshape(n).astype(jnp.int32),
        position.reshape(n).astype(jnp.int32),
    ])
    idx_h = ids.reshape(4, n // CHUNK, CHUNK).transpose(1, 0, 2)
    out = run(idx_h, token_table, segment_table, domain_table, pos_table,
              gamma, beta)
    return out.reshape(b, l, DIM)
